# baseline (speedup 1.0000x reference)
import numpy as np
import jax
import jax.numpy as jnp
from jax.experimental import pallas as pl
from jax.experimental.pallas import tpu as pltpu

N1, N2, N3 = 8, 16, 32
NEG_SLOPE = 0.1
NCLASS = 100
NCLASS_PAD = 128

LB = 32
LBH = LB // 2
LBQ = LB // 4

H1 = W1 = 40
WP1 = 48
H2 = W2 = 20
HP2, WP2 = H2 + 2, 24
H3 = W3 = 10
WP3 = 16
HOUT = 8

HX = H1 + 4
R1 = HX * WP1
B1 = 2 * WP1

LEAD = 8
R2 = LEAD + HP2 * WP2 + LEAD
B2 = LEAD + WP2

P3 = (HOUT - 1) * WP3 + WP3 // 2
QR = H3 * WP3
R3 = 4 * QR + 64

G1, NCH1 = 8, H1 // 8
G2, NCH2 = 10, H2 // 10
CH1, CH2 = G1 * WP1, G2 * WP2

LANES_IN = LB * 3
LANES1 = LB * N1
LANES2 = LBH * N2
LANES3 = LBQ * N3
LANES_OUT = LBQ * NCLASS_PAD

BF16 = jnp.bfloat16


def _leaky(y):
    return jnp.where(y >= 0, y, NEG_SLOPE * y)


def _net_kernel(x_ref,
                w_in_ref, b_in_ref,
                w10_ref, b10_ref, w11_ref, b11_ref, w12_ref, b12_ref,
                w20_ref, b20_ref, w21_ref, b21_ref, w22_ref, b22_ref,
                w3_ref, b3_ref,
                wn1_ref, bnin1_ref, wn2_ref, bnin2_ref,
                wfc_ref, bfc_ref,
                cmask1_ref, cmask2_ref, sel1_ref, sel2_ref, selp_ref,
                o_ref,
                s1a, s1b, s2p, s2a, s2b, s3):
    f32 = jnp.float32

    @pl.when(pl.program_id(0) == 0)
    def _zero():
        for buf in (s1a, s1b):
            buf[0:B1, :] = jnp.zeros((B1, LANES1), f32)
            buf[R1 - B1:R1, :] = jnp.zeros((B1, LANES1), f32)
        for buf in (s2a, s2b):
            head = LEAD + WP2
            buf[0:head, :] = jnp.zeros((head, 2 * LANES2), f32)
            tail0 = LEAD + (HP2 - 1) * WP2
            buf[tail0:R2, :] = jnp.zeros((R2 - tail0, 2 * LANES2), f32)
        s2p[...] = jnp.zeros_like(s2p)
        s3[...] = jnp.zeros_like(s3)

    cmask1 = cmask1_ref[...]
    cmask2 = cmask2_ref[...]

    def conv3x3(load, store, w_ref, b_ref, wp, ch, n_chunks, base0, cmask,
                relu, bl):
        bias = b_ref[...]
        for c in range(n_chunks):
            base = base0 + c * ch
            acc = None
            for t in range(9):
                off = (t // 3 - 1) * wp + (t % 3 - 1)
                xt = load(base + off, ch)
                r = jnp.dot(xt, w_ref[t], preferred_element_type=f32)
                acc = r if acc is None else acc + r
            y = acc + bias
            if relu:
                y = _leaky(y)
            store(base, y * cmask)

    def rw(src, dst):
        return (lambda r, n: src[r:r + n, :],
                lambda r, v: dst.__setitem__(
                    (slice(r, r + v.shape[0]), slice(None)), v))

    ld, st = rw(x_ref, s1a)
    conv3x3(ld, st, w_in_ref, b_in_ref, WP1, CH1, NCH1, B1, cmask1,
            relu=False, bl=16)
    ld, st = rw(s1a, s1b)
    conv3x3(ld, st, w10_ref, b10_ref, WP1, CH1, NCH1, B1, cmask1,
            relu=True, bl=16)
    ld, st = rw(s1b, s1a)
    conv3x3(ld, st, w11_ref, b11_ref, WP1, CH1, NCH1, B1, cmask1,
            relu=True, bl=16)
    ld, st = rw(s1a, s1b)
    conv3x3(ld, st, w12_ref, b12_ref, WP1, CH1, NCH1, B1, cmask1,
            relu=True, bl=16)

    sel1 = sel1_ref[...]
    for yo in range(H2):
        a = (2 * yo + 2) * WP1
        r1 = s1b[a:a + WP1, :]
        r2 = s1b[a + WP1:a + 2 * WP1, :]
        vm = jnp.maximum(r1, r2)
        hm = jnp.maximum(vm[:-1, :], vm[1:, :])
        pooled = jnp.dot(sel1, hm, preferred_element_type=f32)
        d = LEAD + (yo + 1) * WP2 + 1
        s2p[d:d + W2, :] = pooled

    sel2 = sel2_ref[...]
    for h in range(2):
        hi0, hi1 = h * (LBH * N1), (h + 1) * (LBH * N1)
        ho0, ho1 = h * LANES2, (h + 1) * LANES2

        def ld2(src, c0, c1):
            return lambda r, n: src[r:r + n, c0:c1]

        def st2(dst, c0, c1):
            return lambda r, v: dst.__setitem__(
                (slice(r, r + v.shape[0]), slice(c0, c1)), v)

        conv3x3(ld2(s2p, hi0, hi1), st2(s2a, ho0, ho1),
                w20_ref, b20_ref, WP2, CH2, NCH2, B2, cmask2, True, 8)
        conv3x3(ld2(s2a, ho0, ho1), st2(s2b, ho0, ho1),
                w21_ref, b21_ref, WP2, CH2, NCH2, B2, cmask2, True, 8)
        conv3x3(ld2(s2b, ho0, ho1), st2(s2a, ho0, ho1),
                w22_ref, b22_ref, WP2, CH2, NCH2, B2, cmask2, True, 8)

        for yo in range(H3):
            a = LEAD + (2 * yo + 1) * WP2
            r1 = s2a[a:a + WP2, ho0:ho1]
            r2 = s2a[a + WP2:a + 2 * WP2, ho0:ho1]
            vm = jnp.maximum(r1, r2)
            hm = jnp.maximum(vm[:-1, :], vm[1:, :])
            pooled = jnp.dot(sel2, hm, preferred_element_type=f32)
            for j in range(2):
                q = 2 * h + j
                s3[q * QR + yo * WP3:q * QR + yo * WP3 + W3, :] = (
                    pooled[:, j * (LBQ * N2):(j + 1) * (LBQ * N2)])

    wn1 = wn1_ref[...]
    wn2 = wn2_ref[...]
    bn1 = bnin1_ref[...]
    bn2 = bnin2_ref[...]
    b3 = b3_ref[...]
    wfc = wfc_ref[...]
    bfc = bfc_ref[...]
    selp = selp_ref[...]
    for cq in range(2):
        base = cq * 2 * QR
        acc = None
        for t in range(9):
            off = (t // 3) * WP3 + (t % 3)
            xt = s3[base + off:base + off + 2 * QR, :]
            r = jnp.dot(xt, w3_ref[t], preferred_element_type=f32)
            acc = r if acc is None else acc + r
        z = _leaky(acc + b3)
        z = _leaky(jnp.dot(z, wn1, preferred_element_type=f32) + bn1)
        z = _leaky(jnp.dot(z, wn2, preferred_element_type=f32) + bn2)
        pooled = jnp.dot(selp, z, preferred_element_type=f32)
        out = jnp.dot(pooled, wfc, preferred_element_type=f32) + bfc
        o_ref[2 * cq:2 * cq + 2, :] = out[0:2, :].astype(o_ref.dtype)


def kernel(x, bn0_s, bn0_t, w_in, b_in, bn1_s, bn1_t,
           cs1_w0, cs1_b0, cs1_w1, cs1_b1, cs1_w2, cs1_b2,
           cs2_w0, cs2_b0, cs2_w1, cs2_b1, cs2_w2, cs2_b2,
           cs3_w0, cs3_b0,
           w_nin1, b_nin1, w_nin2, b_nin2,
           bn2_s, bn2_t, w_fc, b_fc):
    n, c, h, w = x.shape
    assert (c, h, w) == (3, H1, W1)

    n_pad = ((n + LB - 1) // LB) * LB
    xb = x if n_pad == n else jnp.pad(x, ((0, n_pad - n), (0, 0), (0, 0), (0, 0)))
    G = n_pad // LB

    x4 = xb.astype(jnp.float32) * bn0_s[None, :, None, None] \
        + bn0_t[None, :, None, None]
    x4 = jnp.pad(x4, ((0, 0), (0, 0), (2, 2), (1, WP1 - W1 - 1)))
    xf = x4.reshape(G, LB * 3, R1).transpose(0, 2, 1)

    def bd_taps(w_, nb, scale=None):
        if scale is not None:
            w_ = w_ * scale
        t = w_.reshape(9, w_.shape[2], w_.shape[3]).astype(jnp.float32)
        eye = jnp.eye(nb, dtype=jnp.float32)
        return jnp.einsum('bc,tij->tbicj', eye, t).reshape(
            9, nb * t.shape[1], nb * t.shape[2])

    def bd_mat(w_, nb):
        w_ = w_.astype(jnp.float32)
        eye = jnp.eye(nb, dtype=jnp.float32)
        return jnp.einsum('bc,ij->bicj', eye, w_).reshape(
            nb * w_.shape[0], nb * w_.shape[1])

    def bd_bias(b_, nb):
        return jnp.tile(b_.reshape(1, -1).astype(jnp.float32), (1, nb))

    w_in_bd = bd_taps(w_in, LB, bn1_s)
    b_in_bd = bd_bias(b_in * bn1_s + bn1_t, LB)
    w10, b10 = bd_taps(cs1_w0, LB), bd_bias(cs1_b0, LB)
    w11, b11 = bd_taps(cs1_w1, LB), bd_bias(cs1_b1, LB)
    w12, b12 = bd_taps(cs1_w2, LB), bd_bias(cs1_b2, LB)
    w20, b20 = bd_taps(cs2_w0, LBH), bd_bias(cs2_b0, LBH)
    w21, b21 = bd_taps(cs2_w1, LBH), bd_bias(cs2_b1, LBH)
    w22, b22 = bd_taps(cs2_w2, LBH), bd_bias(cs2_b2, LBH)
    w3_, b3_ = bd_taps(cs3_w0, LBQ), bd_bias(cs3_b0, LBQ)
    wn1, b_n1 = bd_mat(w_nin1, LBQ), bd_bias(b_nin1, LBQ)
    wn2, b_n2 = bd_mat(w_nin2, LBQ), bd_bias(b_nin2, LBQ)
    wfc = bn2_s[:, None] * w_fc
    bfc = bn2_t @ w_fc + b_fc
    wfc = jnp.pad(wfc.astype(jnp.float32), ((0, 0), (0, NCLASS_PAD - NCLASS)))
    bfc = jnp.pad(bfc.astype(jnp.float32), ((0, NCLASS_PAD - NCLASS),))
    wfc_bd, bfc_bd = bd_mat(wfc, LBQ), bd_bias(bfc, LBQ)

    cols1 = np.arange(CH1) % WP1
    cmask1 = jnp.asarray(((cols1 >= 1) & (cols1 <= W1)).astype(np.float32)[:, None])
    cols2 = np.arange(CH2) % WP2
    cmask2 = jnp.asarray(((cols2 >= 1) & (cols2 <= W2)).astype(np.float32)[:, None])
    s1 = np.zeros((W2, WP1 - 1), np.float32)
    s1[np.arange(W2), 2 * np.arange(W2) + 1] = 1.0
    s2 = np.zeros((W3, WP2 - 1), np.float32)
    s2[np.arange(W3), 2 * np.arange(W3) + 1] = 1.0
    sel1, sel2 = jnp.asarray(s1), jnp.asarray(s2)
    sp = np.zeros((8, 2 * QR), np.float32)
    for j in range(2):
        for yy in range(HOUT):
            for xx in range(HOUT):
                sp[j, j * QR + yy * WP3 + xx] = 1.0 / (HOUT * HOUT)
    selp = jnp.asarray(sp)

    args = [xf, w_in_bd, b_in_bd, w10, b10, w11, b11, w12, b12,
            w20, b20, w21, b21, w22, b22, w3_, b3_,
            wn1, b_n1, wn2, b_n2, wfc_bd, bfc_bd,
            cmask1, cmask2, sel1, sel2, selp]

    def const_spec(a):
        nd = a.ndim
        return pl.BlockSpec(a.shape, lambda i, _nd=nd: (0,) * _nd)

    in_specs = [pl.BlockSpec((None, R1, LANES_IN), lambda i: (i, 0, 0))]
    in_specs += [const_spec(a) for a in args[1:]]

    out = pl.pallas_call(
        _net_kernel,
        out_shape=jax.ShapeDtypeStruct((G, 4, LANES_OUT), jnp.float32),
        grid=(G,),
        in_specs=in_specs,
        out_specs=pl.BlockSpec((None, 4, LANES_OUT), lambda i: (i, 0, 0)),
        scratch_shapes=[
            pltpu.VMEM((R1, LANES1), jnp.float32),
            pltpu.VMEM((R1, LANES1), jnp.float32),
            pltpu.VMEM((R2, LANES1), jnp.float32),
            pltpu.VMEM((R2, 2 * LANES2), jnp.float32),
            pltpu.VMEM((R2, 2 * LANES2), jnp.float32),
            pltpu.VMEM((R3, LBQ * N2), jnp.float32),
        ],
        compiler_params=pltpu.CompilerParams(
            dimension_semantics=("arbitrary",),
            vmem_limit_bytes=60 * 1024 * 1024),
    )(*args)

    out = out.reshape(G * LB, NCLASS_PAD)
    return out[:n, :NCLASS]


# speedup vs baseline: 1.0384x; 1.0384x over previous
import numpy as np
import jax
import jax.numpy as jnp
from jax.experimental import pallas as pl
from jax.experimental.pallas import tpu as pltpu

N1, N2, N3 = 8, 16, 32
NEG_SLOPE = 0.1
NCLASS = 100
NCLASS_PAD = 128

LB = 32
LBH = LB // 2
LBQ = LB // 4

H1 = W1 = 40
WP1 = 48
H2 = W2 = 20
HP2, WP2 = H2 + 2, 24
H3 = W3 = 10
WP3 = 16
HOUT = 8

HX = H1 + 4
R1 = HX * WP1
B1 = 2 * WP1

LEAD = 8
R2 = LEAD + HP2 * WP2 + LEAD
B2 = LEAD + WP2

P3 = (HOUT - 1) * WP3 + WP3 // 2
QR = H3 * WP3
R3 = 4 * QR + 64

G1, NCH1 = 8, H1 // 8
G2, NCH2 = 10, H2 // 10
CH1, CH2 = G1 * WP1, G2 * WP2

LANES_IN = LB * 3
LANES1 = LB * N1
LANES2 = LBH * N2
LANES3 = LBQ * N3
LANES_OUT = LBQ * NCLASS_PAD

BF16 = jnp.bfloat16


def _leaky(y):
    return jnp.where(y >= 0, y, NEG_SLOPE * y)


def _net_kernel(x_ref,
                w_in_ref, b_in_ref,
                w10_ref, b10_ref, w11_ref, b11_ref, w12_ref, b12_ref,
                w20_ref, b20_ref, w21_ref, b21_ref, w22_ref, b22_ref,
                w3_ref, b3_ref,
                wn1_ref, bnin1_ref, wn2_ref, bnin2_ref,
                wfc_ref, bfc_ref,
                cmask1_ref, cmask2_ref, sel1_ref, sel2_ref, selp_ref,
                o_ref,
                s1a, s1b, s2p, s2a, s2b, s3):
    f32 = jnp.float32

    @pl.when(pl.program_id(0) == 0)
    def _zero():
        for buf in (s1a, s1b):
            buf[0:B1, :] = jnp.zeros((B1, LANES1), f32)
            buf[R1 - B1:R1, :] = jnp.zeros((B1, LANES1), f32)
        for buf in (s2a, s2b):
            head = LEAD + WP2
            buf[0:head, :] = jnp.zeros((head, 2 * LANES2), f32)
            tail0 = LEAD + (HP2 - 1) * WP2
            buf[tail0:R2, :] = jnp.zeros((R2 - tail0, 2 * LANES2), f32)
        s2p[...] = jnp.zeros_like(s2p)
        s3[...] = jnp.zeros_like(s3)

    cmask1 = cmask1_ref[...]
    cmask2 = cmask2_ref[...]

    def conv3x3(load, store, w_ref, b_ref, wp, ch, n_chunks, base0, cmask,
                relu, bl):
        bias = b_ref[...]
        for c in range(n_chunks):
            base = base0 + c * ch
            acc = None
            for t in range(9):
                off = (t // 3 - 1) * wp + (t % 3 - 1)
                xt = load(base + off, ch)
                r = jnp.dot(xt, w_ref[t], preferred_element_type=f32)
                acc = r if acc is None else acc + r
            y = acc + bias
            if relu:
                y = _leaky(y)
            store(base, y * cmask)

    def rw(src, dst):
        return (lambda r, n: src[r:r + n, :],
                lambda r, v: dst.__setitem__(
                    (slice(r, r + v.shape[0]), slice(None)), v))

    ld, st = rw(x_ref, s1a)
    conv3x3(ld, st, w_in_ref, b_in_ref, WP1, CH1, NCH1, B1, cmask1,
            relu=False, bl=16)
    ld, st = rw(s1a, s1b)
    conv3x3(ld, st, w10_ref, b10_ref, WP1, CH1, NCH1, B1, cmask1,
            relu=True, bl=16)
    ld, st = rw(s1b, s1a)
    conv3x3(ld, st, w11_ref, b11_ref, WP1, CH1, NCH1, B1, cmask1,
            relu=True, bl=16)
    ld, st = rw(s1a, s1b)
    conv3x3(ld, st, w12_ref, b12_ref, WP1, CH1, NCH1, B1, cmask1,
            relu=True, bl=16)

    sel1 = sel1_ref[...]
    for yo in range(H2):
        a = (2 * yo + 2) * WP1
        r1 = s1b[a:a + WP1, :]
        r2 = s1b[a + WP1:a + 2 * WP1, :]
        vm = jnp.maximum(r1, r2)
        hm = jnp.maximum(vm[:-1, :], vm[1:, :])
        pooled = jnp.dot(sel1, hm, preferred_element_type=f32)
        d = LEAD + (yo + 1) * WP2 + 1
        s2p[d:d + W2, :] = pooled

    sel2 = sel2_ref[...]
    for h in range(2):
        hi0, hi1 = h * (LBH * N1), (h + 1) * (LBH * N1)
        ho0, ho1 = h * LANES2, (h + 1) * LANES2

        def ld2(src, c0, c1):
            return lambda r, n: src[r:r + n, c0:c1]

        def st2(dst, c0, c1):
            return lambda r, v: dst.__setitem__(
                (slice(r, r + v.shape[0]), slice(c0, c1)), v)

        conv3x3(ld2(s2p, hi0, hi1), st2(s2a, ho0, ho1),
                w20_ref, b20_ref, WP2, CH2, NCH2, B2, cmask2, True, 8)
        conv3x3(ld2(s2a, ho0, ho1), st2(s2b, ho0, ho1),
                w21_ref, b21_ref, WP2, CH2, NCH2, B2, cmask2, True, 8)
        conv3x3(ld2(s2b, ho0, ho1), st2(s2a, ho0, ho1),
                w22_ref, b22_ref, WP2, CH2, NCH2, B2, cmask2, True, 8)

        for yo in range(H3):
            a = LEAD + (2 * yo + 1) * WP2
            r1 = s2a[a:a + WP2, ho0:ho1]
            r2 = s2a[a + WP2:a + 2 * WP2, ho0:ho1]
            vm = jnp.maximum(r1, r2)
            hm = jnp.maximum(vm[:-1, :], vm[1:, :])
            pooled = jnp.dot(sel2, hm, preferred_element_type=f32)
            for j in range(2):
                q = 2 * h + j
                s3[q * QR + yo * WP3:q * QR + yo * WP3 + W3, :] = (
                    pooled[:, j * (LBQ * N2):(j + 1) * (LBQ * N2)])

    wn1 = wn1_ref[...]
    wn2 = wn2_ref[...]
    bn1 = bnin1_ref[...]
    bn2 = bnin2_ref[...]
    b3 = b3_ref[...]
    wfc = wfc_ref[...]
    bfc = bfc_ref[...]
    selp = selp_ref[...]
    for cq in range(2):
        base = cq * 2 * QR
        acc = None
        for t in range(9):
            off = (t // 3) * WP3 + (t % 3)
            xt = s3[base + off:base + off + 2 * QR, :]
            r = jnp.dot(xt, w3_ref[t], preferred_element_type=f32)
            acc = r if acc is None else acc + r
        z = _leaky(acc + b3)
        z = _leaky(jnp.dot(z, wn1, preferred_element_type=f32) + bn1)
        z = _leaky(jnp.dot(z, wn2, preferred_element_type=f32) + bn2)
        pooled = jnp.dot(selp, z, preferred_element_type=f32)
        out = jnp.dot(pooled, wfc, preferred_element_type=f32) + bfc
        o_ref[2 * cq:2 * cq + 2, :] = out[0:2, :].astype(o_ref.dtype)


def kernel(x, bn0_s, bn0_t, w_in, b_in, bn1_s, bn1_t,
           cs1_w0, cs1_b0, cs1_w1, cs1_b1, cs1_w2, cs1_b2,
           cs2_w0, cs2_b0, cs2_w1, cs2_b1, cs2_w2, cs2_b2,
           cs3_w0, cs3_b0,
           w_nin1, b_nin1, w_nin2, b_nin2,
           bn2_s, bn2_t, w_fc, b_fc):
    n, c, h, w = x.shape
    assert (c, h, w) == (3, H1, W1)

    n_pad = ((n + LB - 1) // LB) * LB
    xb = x if n_pad == n else jnp.pad(x, ((0, n_pad - n), (0, 0), (0, 0), (0, 0)))
    G = n_pad // LB

    x4 = jnp.transpose(xb, (0, 2, 3, 1)).astype(jnp.float32)
    x4 = x4 * bn0_s + bn0_t
    x4 = jnp.pad(x4, ((0, 0), (2, 2), (1, WP1 - W1 - 1), (0, 0)))
    x4 = x4.reshape(n_pad, R1, 3)
    xf = x4.reshape(G, LB, R1, 3).transpose(0, 2, 1, 3).reshape(G, R1, LANES_IN)

    def bd_taps(w_, nb, scale=None):
        if scale is not None:
            w_ = w_ * scale
        t = w_.reshape(9, w_.shape[2], w_.shape[3]).astype(jnp.float32)
        eye = jnp.eye(nb, dtype=jnp.float32)
        return jnp.einsum('bc,tij->tbicj', eye, t).reshape(
            9, nb * t.shape[1], nb * t.shape[2])

    def bd_mat(w_, nb):
        w_ = w_.astype(jnp.float32)
        eye = jnp.eye(nb, dtype=jnp.float32)
        return jnp.einsum('bc,ij->bicj', eye, w_).reshape(
            nb * w_.shape[0], nb * w_.shape[1])

    def bd_bias(b_, nb):
        return jnp.tile(b_.reshape(1, -1).astype(jnp.float32), (1, nb))

    w_in_bd = bd_taps(w_in, LB, bn1_s)
    b_in_bd = bd_bias(b_in * bn1_s + bn1_t, LB)
    w10, b10 = bd_taps(cs1_w0, LB), bd_bias(cs1_b0, LB)
    w11, b11 = bd_taps(cs1_w1, LB), bd_bias(cs1_b1, LB)
    w12, b12 = bd_taps(cs1_w2, LB), bd_bias(cs1_b2, LB)
    w20, b20 = bd_taps(cs2_w0, LBH), bd_bias(cs2_b0, LBH)
    w21, b21 = bd_taps(cs2_w1, LBH), bd_bias(cs2_b1, LBH)
    w22, b22 = bd_taps(cs2_w2, LBH), bd_bias(cs2_b2, LBH)
    w3_, b3_ = bd_taps(cs3_w0, LBQ), bd_bias(cs3_b0, LBQ)
    wn1, b_n1 = bd_mat(w_nin1, LBQ), bd_bias(b_nin1, LBQ)
    wn2, b_n2 = bd_mat(w_nin2, LBQ), bd_bias(b_nin2, LBQ)
    wfc = bn2_s[:, None] * w_fc
    bfc = bn2_t @ w_fc + b_fc
    wfc = jnp.pad(wfc.astype(jnp.float32), ((0, 0), (0, NCLASS_PAD - NCLASS)))
    bfc = jnp.pad(bfc.astype(jnp.float32), ((0, NCLASS_PAD - NCLASS),))
    wfc_bd, bfc_bd = bd_mat(wfc, LBQ), bd_bias(bfc, LBQ)

    cols1 = np.arange(CH1) % WP1
    cmask1 = jnp.asarray(((cols1 >= 1) & (cols1 <= W1)).astype(np.float32)[:, None])
    cols2 = np.arange(CH2) % WP2
    cmask2 = jnp.asarray(((cols2 >= 1) & (cols2 <= W2)).astype(np.float32)[:, None])
    s1 = np.zeros((W2, WP1 - 1), np.float32)
    s1[np.arange(W2), 2 * np.arange(W2) + 1] = 1.0
    s2 = np.zeros((W3, WP2 - 1), np.float32)
    s2[np.arange(W3), 2 * np.arange(W3) + 1] = 1.0
    sel1, sel2 = jnp.asarray(s1), jnp.asarray(s2)
    sp = np.zeros((8, 2 * QR), np.float32)
    for j in range(2):
        for yy in range(HOUT):
            for xx in range(HOUT):
                sp[j, j * QR + yy * WP3 + xx] = 1.0 / (HOUT * HOUT)
    selp = jnp.asarray(sp)

    args = [xf, w_in_bd, b_in_bd, w10, b10, w11, b11, w12, b12,
            w20, b20, w21, b21, w22, b22, w3_, b3_,
            wn1, b_n1, wn2, b_n2, wfc_bd, bfc_bd,
            cmask1, cmask2, sel1, sel2, selp]

    def const_spec(a):
        nd = a.ndim
        return pl.BlockSpec(a.shape, lambda i, _nd=nd: (0,) * _nd)

    in_specs = [pl.BlockSpec((None, R1, LANES_IN), lambda i: (i, 0, 0))]
    in_specs += [const_spec(a) for a in args[1:]]

    out = pl.pallas_call(
        _net_kernel,
        out_shape=jax.ShapeDtypeStruct((G, 4, LANES_OUT), jnp.float32),
        grid=(G,),
        in_specs=in_specs,
        out_specs=pl.BlockSpec((None, 4, LANES_OUT), lambda i: (i, 0, 0)),
        scratch_shapes=[
            pltpu.VMEM((R1, LANES1), jnp.float32),
            pltpu.VMEM((R1, LANES1), jnp.float32),
            pltpu.VMEM((R2, LANES1), jnp.float32),
            pltpu.VMEM((R2, 2 * LANES2), jnp.float32),
            pltpu.VMEM((R2, 2 * LANES2), jnp.float32),
            pltpu.VMEM((R3, LBQ * N2), jnp.float32),
        ],
        compiler_params=pltpu.CompilerParams(
            dimension_semantics=("arbitrary",),
            vmem_limit_bytes=60 * 1024 * 1024),
    )(*args)

    out = out.reshape(G * LB, NCLASS_PAD)
    return out[:n, :NCLASS]


# speedup vs baseline: 1.1215x; 1.0800x over previous
import numpy as np
import jax
import jax.numpy as jnp
from jax.experimental import pallas as pl
from jax.experimental.pallas import tpu as pltpu

N1, N2, N3 = 8, 16, 32
NEG_SLOPE = 0.1
NCLASS = 100
NCLASS_PAD = 128

LB = 32
LBH = LB // 2
LBQ = LB // 4

H1 = W1 = 40
HP1, WP1 = H1 + 2, 48
H2 = W2 = 20
HP2, WP2 = H2 + 2, 24
H3 = W3 = 10
WP3 = 16
HOUT = 8

LEAD = 8
R1 = LEAD + HP1 * WP1 + LEAD
R2 = LEAD + HP2 * WP2 + LEAD
R3 = H3 * WP3

G1, NCH1 = 8, H1 // 8
G2, NCH2 = 10, H2 // 10
CH1, CH2 = G1 * WP1, G2 * WP2
P3 = (HOUT - 1) * WP3 + WP3 // 2

LANES_IN = LB * 3
LANES1 = LB * N1
LANES2 = LBH * N2
LANES3 = LBQ * N3
LANES_OUT = LB * NCLASS_PAD


def _leaky(y):
    return jnp.where(y >= 0, y, NEG_SLOPE * y)


def _net_kernel(x_ref,
                w_in_ref, b_in_ref,
                w10_ref, b10_ref, w11_ref, b11_ref, w12_ref, b12_ref,
                w20_ref, b20_ref, w21_ref, b21_ref, w22_ref, b22_ref,
                w3_ref, b3_ref,
                wn1_ref, bnin1_ref, wn2_ref, bnin2_ref,
                wfc_ref, bfc_ref,
                cmask1_ref, cmask2_ref, vmask3_ref, sel1_ref, sel2_ref,
                o_ref,
                s1a, s1b, s2p, s2a, s2b, s3):
    f32 = jnp.float32

    for buf, wp, hp in ((s1a, WP1, HP1), (s1b, WP1, HP1),
                        (s2a, WP2, HP2), (s2b, WP2, HP2)):
        lanes = buf.shape[1]
        head = LEAD + wp
        buf[0:head, :] = jnp.zeros((head, lanes), f32)
        tail0 = LEAD + (hp - 1) * wp
        buf[tail0:tail0 + wp + LEAD, :] = jnp.zeros((wp + LEAD, lanes), f32)
    s2p[...] = jnp.zeros_like(s2p)
    s3[...] = jnp.zeros_like(s3)

    cmask1 = cmask1_ref[...]
    cmask2 = cmask2_ref[...]

    def conv3x3(load, store, w_ref, b_ref, wp, ch, n_chunks, cmask, relu):
        bias = b_ref[...]
        for c in range(n_chunks):
            base = LEAD + wp + c * ch
            acc = None
            for t in range(9):
                off = (t // 3 - 1) * wp + (t % 3 - 1)
                xt = load(base + off, ch)
                r = jnp.dot(xt, w_ref[t], preferred_element_type=f32)
                acc = r if acc is None else acc + r
            y = acc + bias
            if relu:
                y = _leaky(y)
            store(base, y * cmask)

    conv3x3(lambda r, n: x_ref[r:r + n, :],
            lambda r, v: s1a.__setitem__((slice(r, r + CH1), slice(None)), v),
            w_in_ref, b_in_ref, WP1, CH1, NCH1, cmask1, relu=False)

    def rw(src, dst):
        return (lambda r, n: src[r:r + n, :],
                lambda r, v: dst.__setitem__(
                    (slice(r, r + v.shape[0]), slice(None)), v))

    ld, st = rw(s1a, s1b)
    conv3x3(ld, st, w10_ref, b10_ref, WP1, CH1, NCH1, cmask1, relu=True)
    ld, st = rw(s1b, s1a)
    conv3x3(ld, st, w11_ref, b11_ref, WP1, CH1, NCH1, cmask1, relu=True)
    ld, st = rw(s1a, s1b)
    conv3x3(ld, st, w12_ref, b12_ref, WP1, CH1, NCH1, cmask1, relu=True)

    sel1 = sel1_ref[...]
    for yo in range(H2):
        a = LEAD + (2 * yo + 1) * WP1
        r1 = s1b[a:a + WP1, :]
        r2 = s1b[a + WP1:a + 2 * WP1, :]
        vm = jnp.maximum(r1, r2)
        hm = jnp.maximum(vm[:-1, :], vm[1:, :])
        pooled = jnp.dot(sel1, hm, preferred_element_type=f32)
        d = LEAD + (yo + 1) * WP2 + 1
        s2p[d:d + W2, :] = pooled

    sel2 = sel2_ref[...]
    for h in range(2):
        hi0, hi1 = h * (LBH * N1), (h + 1) * (LBH * N1)
        ho0, ho1 = h * LANES2, (h + 1) * LANES2

        def ld2(src, c0, c1):
            return lambda r, n: src[r:r + n, c0:c1]

        def st2(dst, c0, c1):
            return lambda r, v: dst.__setitem__(
                (slice(r, r + v.shape[0]), slice(c0, c1)), v)

        conv3x3(ld2(s2p, hi0, hi1), st2(s2a, ho0, ho1),
                w20_ref, b20_ref, WP2, CH2, NCH2, cmask2, relu=True)
        conv3x3(ld2(s2a, ho0, ho1), st2(s2b, ho0, ho1),
                w21_ref, b21_ref, WP2, CH2, NCH2, cmask2, relu=True)
        conv3x3(ld2(s2b, ho0, ho1), st2(s2a, ho0, ho1),
                w22_ref, b22_ref, WP2, CH2, NCH2, cmask2, relu=True)

        for yo in range(H3):
            a = LEAD + (2 * yo + 1) * WP2
            r1 = s2a[a:a + WP2, ho0:ho1]
            r2 = s2a[a + WP2:a + 2 * WP2, ho0:ho1]
            vm = jnp.maximum(r1, r2)
            hm = jnp.maximum(vm[:-1, :], vm[1:, :])
            pooled = jnp.dot(sel2, hm, preferred_element_type=f32)
            s3[yo * WP3:yo * WP3 + W3, ho0:ho1] = pooled

    vmask3 = vmask3_ref[...]
    wn1 = wn1_ref[...]
    wn2 = wn2_ref[...]
    bn1 = bnin1_ref[...]
    bn2 = bnin2_ref[...]
    wfc = wfc_ref[...]
    bfc = bfc_ref[...]
    for q in range(4):
        qi0, qi1 = q * (LBQ * N2), (q + 1) * (LBQ * N2)
        acc = None
        for t in range(9):
            off = (t // 3) * WP3 + (t % 3)
            xt = s3[off:off + P3, qi0:qi1]
            r = jnp.dot(xt, w3_ref[t], preferred_element_type=f32)
            acc = r if acc is None else acc + r
        z = _leaky(acc + b3_ref[...])
        z = _leaky(jnp.dot(z, wn1, preferred_element_type=f32) + bn1)
        z = _leaky(jnp.dot(z, wn2, preferred_element_type=f32) + bn2)
        pooled = jnp.sum(z * vmask3, axis=0, keepdims=True) * (1.0 / (HOUT * HOUT))
        out = jnp.dot(pooled, wfc, preferred_element_type=f32) + bfc
        o_ref[0:1, q * (LBQ * NCLASS_PAD):(q + 1) * (LBQ * NCLASS_PAD)] = (
            out.astype(o_ref.dtype))


def kernel(x, bn0_s, bn0_t, w_in, b_in, bn1_s, bn1_t,
           cs1_w0, cs1_b0, cs1_w1, cs1_b1, cs1_w2, cs1_b2,
           cs2_w0, cs2_b0, cs2_w1, cs2_b1, cs2_w2, cs2_b2,
           cs3_w0, cs3_b0,
           w_nin1, b_nin1, w_nin2, b_nin2,
           bn2_s, bn2_t, w_fc, b_fc):
    n, c, h, w = x.shape
    assert (c, h, w) == (3, H1, W1)

    n_pad = ((n + LB - 1) // LB) * LB
    xb = x if n_pad == n else jnp.pad(x, ((0, n_pad - n), (0, 0), (0, 0), (0, 0)))
    G = n_pad // LB

    xf = jnp.transpose(xb, (0, 2, 3, 1)).astype(jnp.float32)
    xf = xf * bn0_s + bn0_t
    xf = jnp.pad(xf, ((0, 0), (1, 1), (1, WP1 - W1 - 1), (0, 0)))
    xf = xf.reshape(n_pad, HP1 * WP1, 3)
    xf = jnp.pad(xf, ((0, 0), (LEAD, LEAD), (0, 0)))
    xf = xf.reshape(G, LB, R1, 3).transpose(0, 2, 1, 3).reshape(G, R1, LANES_IN)

    def bd_taps(w_, nb, scale=None):
        if scale is not None:
            w_ = w_ * scale
        t = w_.reshape(9, w_.shape[2], w_.shape[3]).astype(jnp.float32)
        eye = jnp.eye(nb, dtype=jnp.float32)
        return jnp.einsum('bc,tij->tbicj', eye, t).reshape(
            9, nb * t.shape[1], nb * t.shape[2])

    def bd_mat(w_, nb):
        w_ = w_.astype(jnp.float32)
        eye = jnp.eye(nb, dtype=jnp.float32)
        return jnp.einsum('bc,ij->bicj', eye, w_).reshape(
            nb * w_.shape[0], nb * w_.shape[1])

    def bd_bias(b_, nb):
        return jnp.tile(b_.reshape(1, -1).astype(jnp.float32), (1, nb))

    w_in_bd = bd_taps(w_in, LB, bn1_s)
    b_in_bd = bd_bias(b_in * bn1_s + bn1_t, LB)
    w10, b10 = bd_taps(cs1_w0, LB), bd_bias(cs1_b0, LB)
    w11, b11 = bd_taps(cs1_w1, LB), bd_bias(cs1_b1, LB)
    w12, b12 = bd_taps(cs1_w2, LB), bd_bias(cs1_b2, LB)
    w20, b20 = bd_taps(cs2_w0, LBH), bd_bias(cs2_b0, LBH)
    w21, b21 = bd_taps(cs2_w1, LBH), bd_bias(cs2_b1, LBH)
    w22, b22 = bd_taps(cs2_w2, LBH), bd_bias(cs2_b2, LBH)
    w3_, b3_ = bd_taps(cs3_w0, LBQ), bd_bias(cs3_b0, LBQ)
    wn1, b_n1 = bd_mat(w_nin1, LBQ), bd_bias(b_nin1, LBQ)
    wn2, b_n2 = bd_mat(w_nin2, LBQ), bd_bias(b_nin2, LBQ)
    wfc = bn2_s[:, None] * w_fc
    bfc = bn2_t @ w_fc + b_fc
    wfc = jnp.pad(wfc.astype(jnp.float32), ((0, 0), (0, NCLASS_PAD - NCLASS)))
    bfc = jnp.pad(bfc.astype(jnp.float32), ((0, NCLASS_PAD - NCLASS),))
    wfc_bd, bfc_bd = bd_mat(wfc, LBQ), bd_bias(bfc, LBQ)

    cols1 = np.arange(CH1) % WP1
    cmask1 = jnp.asarray(((cols1 >= 1) & (cols1 <= W1)).astype(np.float32)[:, None])
    cols2 = np.arange(CH2) % WP2
    cmask2 = jnp.asarray(((cols2 >= 1) & (cols2 <= W2)).astype(np.float32)[:, None])
    cols3 = np.arange(P3) % WP3
    vmask3 = jnp.asarray((cols3 < HOUT).astype(np.float32)[:, None])
    s1 = np.zeros((W2, WP1 - 1), np.float32)
    s1[np.arange(W2), 2 * np.arange(W2) + 1] = 1.0
    s2 = np.zeros((W3, WP2 - 1), np.float32)
    s2[np.arange(W3), 2 * np.arange(W3) + 1] = 1.0
    sel1, sel2 = jnp.asarray(s1), jnp.asarray(s2)

    args = [xf, w_in_bd, b_in_bd, w10, b10, w11, b11, w12, b12,
            w20, b20, w21, b21, w22, b22, w3_, b3_,
            wn1, b_n1, wn2, b_n2, wfc_bd, bfc_bd,
            cmask1, cmask2, vmask3, sel1, sel2]

    def const_spec(a):
        nd = a.ndim
        return pl.BlockSpec(a.shape, lambda i, _nd=nd: (0,) * _nd)

    in_specs = [pl.BlockSpec((None, R1, LANES_IN), lambda i: (i, 0, 0))]
    in_specs += [const_spec(a) for a in args[1:]]

    out = pl.pallas_call(
        _net_kernel,
        out_shape=jax.ShapeDtypeStruct((G, 1, LANES_OUT), jnp.float32),
        grid=(G,),
        in_specs=in_specs,
        out_specs=pl.BlockSpec((None, 1, LANES_OUT), lambda i: (i, 0, 0)),
        scratch_shapes=[
            pltpu.VMEM((R1, LANES1), jnp.float32),
            pltpu.VMEM((R1, LANES1), jnp.float32),
            pltpu.VMEM((R2, LANES1), jnp.float32),
            pltpu.VMEM((R2, 2 * LANES2), jnp.float32),
            pltpu.VMEM((R2, 2 * LANES2), jnp.float32),
            pltpu.VMEM((R3, 2 * LANES2), jnp.float32),
        ],
        compiler_params=pltpu.CompilerParams(
            dimension_semantics=("parallel",),
            vmem_limit_bytes=60 * 1024 * 1024),
    )(*args)

    out = out.reshape(G * LB, NCLASS_PAD)
    return out[:n, :NCLASS]
